# baseline (speedup 1.0000x reference)
import functools

import jax
import jax.numpy as jnp
from jax import lax
from jax.experimental import pallas as pl
from jax.experimental.pallas import tpu as pltpu
from jax.experimental.pallas import tpu_sc as plsc

N = 100000
D = 128
H = 128
NC = 16
ALPHA = 0.2

NUM_WORKERS = 32
CHUNK = 112
NUM_SC_CORES = 2
K0 = 28
NPS = NUM_WORKERS * K0 * CHUNK
RING = 4

TC_ROWS = 19200


def _sc_gather(features, i1, i2):
  mesh = plsc.VectorSubcoreMesh(core_axis_name="c", subcore_axis_name="s",
                                num_cores=NUM_SC_CORES)

  @functools.partial(
      pl.kernel,
      out_type=(
          jax.ShapeDtypeStruct((NPS, D), jnp.float32),
          jax.ShapeDtypeStruct((NPS, D), jnp.float32),
      ),
      mesh=mesh,
      scratch_types=[
          pltpu.VMEM((K0 * CHUNK,), jnp.int32),
          pltpu.VMEM((K0 * CHUNK,), jnp.int32),
          pltpu.VMEM((RING, CHUNK, D), jnp.float32),
          pltpu.VMEM((RING, CHUNK, D), jnp.float32),
          pltpu.SemaphoreType.DMA((RING,)),
          pltpu.SemaphoreType.DMA((RING,)),
          pltpu.SemaphoreType.DMA((RING,)),
          pltpu.SemaphoreType.DMA((RING,)),
      ],
  )
  def gather_kernel(f_hbm, i1_hbm, i2_hbm, g1_hbm, g2_hbm,
                    idx1_v, idx2_v, buf1, buf2, gs1, gs2, ws1, ws2):
    cid = lax.axis_index("c")
    sid = lax.axis_index("s")
    wid = cid * 16 + sid
    kcount = K0
    cstart = wid * K0

    def fire_gather(k, b):
      pltpu.async_copy(f_hbm.at[idx1_v.at[pl.ds(k * CHUNK, CHUNK)]],
                       buf1.at[b], gs1.at[b])
      pltpu.async_copy(f_hbm.at[idx2_v.at[pl.ds(k * CHUNK, CHUNK)]],
                       buf2.at[b], gs2.at[b])

    row0 = pl.multiple_of(cstart * CHUNK, CHUNK)
    pltpu.sync_copy(i1_hbm.at[pl.ds(row0, K0 * CHUNK)], idx1_v)
    pltpu.sync_copy(i2_hbm.at[pl.ds(row0, K0 * CHUNK)], idx2_v)
    for b in range(RING):
      fire_gather(b, b)

    def wait_write(b):
      pltpu.make_async_copy(buf1.at[b], g1_hbm.at[pl.ds(0, CHUNK)],
                            ws1.at[b]).wait()
      pltpu.make_async_copy(buf2.at[b], g2_hbm.at[pl.ds(0, CHUNK)],
                            ws2.at[b]).wait()

    def body(j, carry):
      b = lax.rem(j, RING)
      off = pl.multiple_of((cstart + j) * CHUNK, 16)
      pltpu.make_async_copy(f_hbm.at[pl.ds(0, CHUNK)], buf1.at[b],
                            gs1.at[b]).wait()
      pltpu.async_copy(buf1.at[b], g1_hbm.at[pl.ds(off, CHUNK)], ws1.at[b])
      pltpu.make_async_copy(f_hbm.at[pl.ds(0, CHUNK)], buf2.at[b],
                            gs2.at[b]).wait()
      pltpu.async_copy(buf2.at[b], g2_hbm.at[pl.ds(off, CHUNK)], ws2.at[b])

      @pl.when(jnp.logical_and(j >= 1, j - 1 + RING < kcount))
      def _refill():
        b_prev = lax.rem(j - 1, RING)
        wait_write(b_prev)
        fire_gather(j - 1 + RING, b_prev)

      return carry

    lax.fori_loop(0, kcount, body, 0)

    for b in range(RING):
      wait_write(b)

  return gather_kernel(features, i1, i2)


def _tc_body(g1_ref, g2_ref, c_ref, w_ref, v_ref, o_ref, a_ref, b_ref):
  @pl.when(pl.program_id(0) == 0)
  def _fold_weights():
    cw1 = jnp.dot(c_ref[...], w_ref[:H, :], preferred_element_type=jnp.float32)
    cw2 = jnp.dot(c_ref[...], w_ref[H:, :], preferred_element_type=jnp.float32)
    a_ref[...] = jnp.dot(cw1, v_ref[...], preferred_element_type=jnp.float32)
    b_ref[...] = jnp.dot(cw2, v_ref[...], preferred_element_type=jnp.float32)

  outs = (jnp.dot(g1_ref[...], a_ref[...], preferred_element_type=jnp.float32)
          + jnp.dot(g2_ref[...], b_ref[...], preferred_element_type=jnp.float32))
  r = jnp.where(outs >= 0, outs, ALPHA * outs)
  m = jnp.max(r, axis=1, keepdims=True)
  shifted = r - m
  res = shifted - jnp.log(jnp.sum(jnp.exp(shifted), axis=1, keepdims=True))
  o_ref[...] = res.T


def _tc_fused(g1, g2, C, W, V):
  return pl.pallas_call(
      _tc_body,
      grid=(pl.cdiv(N, TC_ROWS),),
      in_specs=[
          pl.BlockSpec((TC_ROWS, D), lambda i: (i, 0)),
          pl.BlockSpec((TC_ROWS, D), lambda i: (i, 0)),
          pl.BlockSpec((D, H), lambda i: (0, 0)),
          pl.BlockSpec((2 * H, H), lambda i: (0, 0)),
          pl.BlockSpec((H, NC), lambda i: (0, 0)),
      ],
      out_specs=pl.BlockSpec((NC, TC_ROWS), lambda i: (0, i)),
      out_shape=jax.ShapeDtypeStruct((NC, N), jnp.float32),
      scratch_shapes=[
          pltpu.VMEM((H, NC), jnp.float32),
          pltpu.VMEM((H, NC), jnp.float32),
      ],
  )(g1, g2, C, W, V).T


def kernel(features, C, W, V, n1, n2):
  def pack(idx):
    pad = jnp.arange(NPS - N, dtype=jnp.int32)
    return jnp.concatenate([idx.astype(jnp.int32), pad])

  g1, g2 = _sc_gather(features, pack(n1), pack(n2))
  return _tc_fused(g1, g2, C, W, V)


# speedup vs baseline: 1.0475x; 1.0475x over previous
import functools

import jax
import jax.numpy as jnp
from jax import lax
from jax.experimental import pallas as pl
from jax.experimental.pallas import tpu as pltpu
from jax.experimental.pallas import tpu_sc as plsc

N = 100000
D = 128
H = 128
NC = 16
ALPHA = 0.2

NUM_WORKERS = 32
CHUNK = 112
NUM_SC_CORES = 2
K0 = 28
NPS = NUM_WORKERS * K0 * CHUNK
RING = 4

TC_ROWS = 10240


def _sc_gather(features, i1, i2):
  mesh = plsc.VectorSubcoreMesh(core_axis_name="c", subcore_axis_name="s",
                                num_cores=NUM_SC_CORES)

  @functools.partial(
      pl.kernel,
      out_type=(
          jax.ShapeDtypeStruct((NPS, D), jnp.float32),
          jax.ShapeDtypeStruct((NPS, D), jnp.float32),
      ),
      mesh=mesh,
      scratch_types=[
          pltpu.VMEM((K0 * CHUNK,), jnp.int32),
          pltpu.VMEM((K0 * CHUNK,), jnp.int32),
          pltpu.VMEM((RING, CHUNK, D), jnp.float32),
          pltpu.VMEM((RING, CHUNK, D), jnp.float32),
          pltpu.SemaphoreType.DMA((RING,)),
          pltpu.SemaphoreType.DMA((RING,)),
          pltpu.SemaphoreType.DMA((RING,)),
          pltpu.SemaphoreType.DMA((RING,)),
      ],
  )
  def gather_kernel(f_hbm, i1_hbm, i2_hbm, g1_hbm, g2_hbm,
                    idx1_v, idx2_v, buf1, buf2, gs1, gs2, ws1, ws2):
    cid = lax.axis_index("c")
    sid = lax.axis_index("s")
    wid = cid * 16 + sid
    kcount = K0
    cstart = wid * K0

    def fire_gather(k, b):
      pltpu.async_copy(f_hbm.at[idx1_v.at[pl.ds(k * CHUNK, CHUNK)]],
                       buf1.at[b], gs1.at[b])
      pltpu.async_copy(f_hbm.at[idx2_v.at[pl.ds(k * CHUNK, CHUNK)]],
                       buf2.at[b], gs2.at[b])

    row0 = pl.multiple_of(cstart * CHUNK, CHUNK)
    pltpu.sync_copy(i1_hbm.at[pl.ds(row0, K0 * CHUNK)], idx1_v)
    pltpu.sync_copy(i2_hbm.at[pl.ds(row0, K0 * CHUNK)], idx2_v)
    for b in range(RING):
      fire_gather(b, b)

    def wait_write(b):
      pltpu.make_async_copy(buf1.at[b], g1_hbm.at[pl.ds(0, CHUNK)],
                            ws1.at[b]).wait()
      pltpu.make_async_copy(buf2.at[b], g2_hbm.at[pl.ds(0, CHUNK)],
                            ws2.at[b]).wait()

    def body(j, carry):
      b = lax.rem(j, RING)
      off = pl.multiple_of((cstart + j) * CHUNK, 16)
      pltpu.make_async_copy(f_hbm.at[pl.ds(0, CHUNK)], buf1.at[b],
                            gs1.at[b]).wait()
      pltpu.async_copy(buf1.at[b], g1_hbm.at[pl.ds(off, CHUNK)], ws1.at[b])
      pltpu.make_async_copy(f_hbm.at[pl.ds(0, CHUNK)], buf2.at[b],
                            gs2.at[b]).wait()
      pltpu.async_copy(buf2.at[b], g2_hbm.at[pl.ds(off, CHUNK)], ws2.at[b])

      @pl.when(jnp.logical_and(j >= 1, j - 1 + RING < kcount))
      def _refill():
        b_prev = lax.rem(j - 1, RING)
        wait_write(b_prev)
        fire_gather(j - 1 + RING, b_prev)

      return carry

    lax.fori_loop(0, kcount, body, 0)

    for b in range(RING):
      wait_write(b)

  return gather_kernel(features, i1, i2)


def _tc_body(g1_ref, g2_ref, c_ref, w_ref, v_ref, o_ref, a_ref, b_ref):
  @pl.when(pl.program_id(0) == 0)
  def _fold_weights():
    cw1 = jnp.dot(c_ref[...], w_ref[:H, :], preferred_element_type=jnp.float32)
    cw2 = jnp.dot(c_ref[...], w_ref[H:, :], preferred_element_type=jnp.float32)
    a_ref[...] = jnp.dot(cw1, v_ref[...], preferred_element_type=jnp.float32)
    b_ref[...] = jnp.dot(cw2, v_ref[...], preferred_element_type=jnp.float32)

  outs = (jnp.dot(g1_ref[...], a_ref[...], preferred_element_type=jnp.float32)
          + jnp.dot(g2_ref[...], b_ref[...], preferred_element_type=jnp.float32))
  r = jnp.where(outs >= 0, outs, ALPHA * outs)
  m = jnp.max(r, axis=1, keepdims=True)
  shifted = r - m
  res = shifted - jnp.log(jnp.sum(jnp.exp(shifted), axis=1, keepdims=True))
  o_ref[...] = res.T


def _tc_fused(g1, g2, C, W, V):
  return pl.pallas_call(
      _tc_body,
      grid=(pl.cdiv(N, TC_ROWS),),
      in_specs=[
          pl.BlockSpec((TC_ROWS, D), lambda i: (i, 0)),
          pl.BlockSpec((TC_ROWS, D), lambda i: (i, 0)),
          pl.BlockSpec((D, H), lambda i: (0, 0)),
          pl.BlockSpec((2 * H, H), lambda i: (0, 0)),
          pl.BlockSpec((H, NC), lambda i: (0, 0)),
      ],
      out_specs=pl.BlockSpec((NC, TC_ROWS), lambda i: (0, i)),
      out_shape=jax.ShapeDtypeStruct((NC, N), jnp.float32),
      scratch_shapes=[
          pltpu.VMEM((H, NC), jnp.float32),
          pltpu.VMEM((H, NC), jnp.float32),
      ],
  )(g1, g2, C, W, V).T


def kernel(features, C, W, V, n1, n2):
  def pack(idx):
    pad = jnp.arange(NPS - N, dtype=jnp.int32)
    return jnp.concatenate([idx.astype(jnp.int32), pad])

  g1, g2 = _sc_gather(features, pack(n1), pack(n2))
  return _tc_fused(g1, g2, C, W, V)
